# baseline (speedup 1.0000x reference)
import functools
import jax
import jax.numpy as jnp
from jax import lax
from jax.experimental import pallas as pl
from jax.experimental.pallas import tpu as pltpu
from jax.experimental.pallas import tpu_sc as plsc

N = 10000
E = 320000
CH = 128
KAPPA = 0.95

NC, NS = 2, 16
NW = NC * NS
C = 128
NCHUNK = 80
EPAD = NW * NCHUNK * C
NP = 10112
RPS = NP // NS

BR = 1000


def _tc1_body(x_ref, emb_ref, W1_ref, F_ref, u1_ref, e2_ref, wp_ref):
    @pl.when(pl.program_id(0) == 0)
    def _():
        Fm = F_ref[...]
        Wp = lax.dot_general(Fm, Fm, (((0,), (0,)), ((), ())),
                             preferred_element_type=jnp.float32)
        nrm = jnp.sqrt(jnp.sum(Wp * Wp))
        Wp = jnp.where(nrm > 1.0, Wp / (nrm + 1e-5), Wp)
        wp_ref[...] = Wp * KAPPA
    u1_ref[...] = jnp.dot(x_ref[...], W1_ref[...],
                          preferred_element_type=jnp.float32)
    e2_ref[...] = jnp.dot(emb_ref[...], wp_ref[...],
                          preferred_element_type=jnp.float32)


_tc1 = pl.pallas_call(
    _tc1_body,
    grid=(N // BR,),
    in_specs=[
        pl.BlockSpec((BR, CH), lambda i: (i, 0)),
        pl.BlockSpec((BR, CH), lambda i: (i, 0)),
        pl.BlockSpec((CH, CH), lambda i: (0, 0)),
        pl.BlockSpec((CH, CH), lambda i: (0, 0)),
    ],
    out_specs=[pl.BlockSpec((BR, CH), lambda i: (i, 0))] * 2,
    out_shape=[jax.ShapeDtypeStruct((N, CH), jnp.float32)] * 2,
    scratch_shapes=[pltpu.VMEM((CH, CH), jnp.float32)],
)


def _tc2_body(p0_ref, p1_ref, b1_ref, W2_ref, e2_ref, u2_ref):
    h = jnp.maximum(p0_ref[...] + p1_ref[...] + b1_ref[...], 0.0)
    u2_ref[...] = jnp.dot(h, W2_ref[...],
                          preferred_element_type=jnp.float32) + e2_ref[...]


_tc2 = pl.pallas_call(
    _tc2_body,
    grid=(N // BR,),
    in_specs=[
        pl.BlockSpec((BR, CH), lambda i: (i, 0)),
        pl.BlockSpec((BR, CH), lambda i: (i, 0)),
        pl.BlockSpec((1, CH), lambda i: (0, 0)),
        pl.BlockSpec((CH, CH), lambda i: (0, 0)),
        pl.BlockSpec((BR, CH), lambda i: (i, 0)),
    ],
    out_specs=pl.BlockSpec((BR, CH), lambda i: (i, 0)),
    out_shape=jax.ShapeDtypeStruct((N, CH), jnp.float32),
)


def _tc3_body(q0_ref, q1_ref, b2_ref, out_ref):
    out_ref[...] = q0_ref[...] + q1_ref[...] + b2_ref[...]


_tc3 = pl.pallas_call(
    _tc3_body,
    grid=(N // BR,),
    in_specs=[
        pl.BlockSpec((BR, CH), lambda i: (i, 0)),
        pl.BlockSpec((BR, CH), lambda i: (i, 0)),
        pl.BlockSpec((1, CH), lambda i: (0, 0)),
    ],
    out_specs=pl.BlockSpec((BR, CH), lambda i: (i, 0)),
    out_shape=jax.ShapeDtypeStruct((N, CH), jnp.float32),
)


_sc_mesh = plsc.VectorSubcoreMesh(
    core_axis_name="c", subcore_axis_name="s", num_cores=NC, num_subcores=NS)


@functools.partial(
    pl.kernel,
    out_type=jax.ShapeDtypeStruct((NC, NP, CH), jnp.float32),
    mesh=_sc_mesh,
    scratch_types=[
        pltpu.VMEM((NCHUNK, C), jnp.int32),
        pltpu.VMEM((C,), jnp.int32),
        pltpu.VMEM((C,), jnp.int32),
        pltpu.VMEM((C,), jnp.int32),
        pltpu.VMEM((C,), jnp.int32),
        pltpu.VMEM((C, CH), jnp.float32),
        pltpu.VMEM((C, CH), jnp.float32),
        pltpu.VMEM_SHARED((NP, CH), jnp.float32),
        pltpu.SemaphoreType.DMA,
        pltpu.SemaphoreType.DMA,
    ],
)
def _segsum(u_hbm, pk_hbm, zeros_hbm, out_hbm,
            pk_v, src0, dst0, src1, dst1, buf0, buf1, acc, sem0, sem1):
    cid = lax.axis_index("c")
    sid = lax.axis_index("s")
    wid = sid * NC + cid
    pltpu.sync_copy(zeros_hbm.at[pl.ds(sid * RPS, RPS)],
                    acc.at[pl.ds(sid * RPS, RPS)])
    pltpu.sync_copy(pk_hbm.at[wid], pk_v)
    plsc.subcore_barrier()

    def unpack(j, src_c, dst_c):
        for k in range(C // 16):
            p = pk_v[j, pl.ds(k * 16, 16)]
            src_c[pl.ds(k * 16, 16)] = lax.bitwise_and(p, 0xFFFF)
            dst_c[pl.ds(k * 16, 16)] = lax.shift_right_logical(p, 16)

    unpack(0, src0, dst0)
    pltpu.async_copy(u_hbm.at[src0], buf0, sem0)
    unpack(1, src1, dst1)
    pltpu.async_copy(u_hbm.at[src1], buf1, sem1)

    def body(i, carry):
        j = 2 * i
        pltpu.make_async_copy(u_hbm.at[src0], buf0, sem0).wait()
        pltpu.sync_copy(buf0, acc.at[dst0], add=True)

        @pl.when(j + 2 < NCHUNK)
        def _():
            unpack(j + 2, src0, dst0)
            pltpu.async_copy(u_hbm.at[src0], buf0, sem0)

        pltpu.make_async_copy(u_hbm.at[src1], buf1, sem1).wait()
        pltpu.sync_copy(buf1, acc.at[dst1], add=True)

        @pl.when(j + 3 < NCHUNK)
        def _():
            unpack(j + 3, src1, dst1)
            pltpu.async_copy(u_hbm.at[src1], buf1, sem1)

        return carry

    lax.fori_loop(0, NCHUNK // 2, body, 0)
    plsc.subcore_barrier()
    pltpu.sync_copy(acc.at[pl.ds(sid * RPS, RPS)],
                    out_hbm.at[cid, pl.ds(sid * RPS, RPS)])


def kernel(x, edge_index, W1, b1, W2, b2, F, emb):
    pad = EPAD - E
    src = jnp.concatenate([edge_index[0], jnp.zeros((pad,), jnp.int32)])
    dst = jnp.concatenate(
        [edge_index[1], N + (jnp.arange(pad, dtype=jnp.int32) % (NP - N))])
    packed = jnp.bitwise_or(src, jnp.left_shift(dst, 16)).reshape(NW, NCHUNK, C)
    zeros = jnp.zeros((NP, CH), jnp.float32)
    u1, e2 = _tc1(x, emb, W1, F)
    p = _segsum(u1, packed, zeros)
    u2 = _tc2(p[0, :N], p[1, :N], b1.reshape(1, CH), W2, e2)
    q = _segsum(u2, packed, zeros)
    return _tc3(q[0, :N], q[1, :N], b2.reshape(1, CH))
